# baseline (speedup 1.0000x reference)
import functools

import jax
import jax.numpy as jnp
import numpy as np
from jax import lax
from jax.experimental import pallas as pl
from jax.experimental.pallas import tpu as pltpu
from jax.experimental.pallas import tpu_sc as plsc

FR = 5
NPF = 32
TEMP = 10000.0
MM = 1_000_000
BB = 16384
NC, NS = 2, 16
NW = NC * NS
QW = BB // NW
WW = BB // NS
ROUNDS = 3
TAG_LEN = NC * MM + 128
DUMMY = NC * MM
NCOLS = FR * 4 * NPF
QQ = 128


def _sc_body(comb_hbm, det2_hbm, w_hbm, q_hbm,
             xout_hbm, dout_hbm, tc_hbm, tag_hbm,
             w_v, woff_v, val_v, s_v, idx2_v,
             q_v, qoff_v, t_v, tc_v, qrow_v, trow_v,
             xq_v, dq_v, sem):
  cid = lax.axis_index("c")
  sid = lax.axis_index("s")
  wid = sid * NC + cid
  coff = cid * MM

  pltpu.sync_copy(w_hbm.at[pl.ds(sid * WW, WW)], w_v)
  for j in range(WW // 16):
    s = pl.ds(j * 16, 16)
    woff_v[s] = w_v[s] + coff
    val_v[s] = lax.iota(jnp.int32, 16) + (sid * WW + j * 16)
  pltpu.async_copy(val_v, tag_hbm.at[woff_v], sem).wait()
  plsc.subcore_barrier()

  for _ in range(ROUNDS):
    pltpu.async_copy(tag_hbm.at[woff_v], s_v, sem).wait()
    for j in range(WW // 16):
      s = pl.ds(j * 16, 16)
      loser = val_v[s] > s_v[s]
      idx2_v[s] = jnp.where(loser, woff_v[s], DUMMY)
    pltpu.async_copy(val_v, tag_hbm.at[idx2_v], sem).wait()
    plsc.subcore_barrier()

  qbase = wid * QW
  pltpu.sync_copy(q_hbm.at[pl.ds(qbase, QW)], q_v)
  for j in range(QW // 16):
    s = pl.ds(j * 16, 16)
    qoff_v[s] = q_v[s] + coff
  pltpu.async_copy(tag_hbm.at[qoff_v], t_v, sem).wait()
  for j in range(QW // 16):
    s = pl.ds(j * 16, 16)
    t = jnp.minimum(jnp.maximum(t_v[s], 0), BB - 1)
    tc_v[s] = t
    qrow_v[s] = lax.shift_right_logical(q_v[s], 2)
    trow_v[s] = lax.shift_right_logical(t, 2)
  pltpu.sync_copy(tc_v, tc_hbm.at[pl.ds(qbase, QW)])
  for b in range(QW // QQ):
    sb = pl.ds(b * QQ, QQ)
    cp_x = pltpu.async_copy(comb_hbm.at[qrow_v.at[sb]], xq_v, sem)
    cp_d = pltpu.async_copy(det2_hbm.at[trow_v.at[sb]], dq_v, sem)
    cp_x.wait()
    pltpu.sync_copy(xq_v, xout_hbm.at[pl.ds(qbase + b * QQ, QQ)])
    cp_d.wait()
    pltpu.sync_copy(dq_v, dout_hbm.at[pl.ds(qbase + b * QQ, QQ)])


@functools.lru_cache(maxsize=None)
def _sc_stage():
  mesh = plsc.VectorSubcoreMesh(core_axis_name="c", subcore_axis_name="s",
                                num_cores=NC, num_subcores=NS)
  return pl.kernel(
      _sc_body,
      out_type=(
          jax.ShapeDtypeStruct((BB, 128), jnp.float32),
          jax.ShapeDtypeStruct((BB, 128), jnp.float32),
          jax.ShapeDtypeStruct((BB,), jnp.int32),
          jax.ShapeDtypeStruct((TAG_LEN,), jnp.int32),
      ),
      mesh=mesh,
      scratch_types=[
          pltpu.VMEM((WW,), jnp.int32),
          pltpu.VMEM((WW,), jnp.int32),
          pltpu.VMEM((WW,), jnp.int32),
          pltpu.VMEM((WW,), jnp.int32),
          pltpu.VMEM((WW,), jnp.int32),
          pltpu.VMEM((QW,), jnp.int32),
          pltpu.VMEM((QW,), jnp.int32),
          pltpu.VMEM((QW,), jnp.int32),
          pltpu.VMEM((QW,), jnp.int32),
          pltpu.VMEM((QW,), jnp.int32),
          pltpu.VMEM((QW,), jnp.int32),
          pltpu.VMEM((QQ, 128), jnp.float32),
          pltpu.VMEM((QQ, 128), jnp.float32),
          pltpu.SemaphoreType.DMA,
      ],
  )


def _sel4(m, a):
  return jnp.where(m == 0, a[:, 0:32],
                   jnp.where(m == 1, a[:, 32:64],
                             jnp.where(m == 2, a[:, 64:96], a[:, 96:128])))


def _tc_body(x_ref, d_ref, q_ref, t_ref, e_ref, t1_ref, coef_ref,
             shift_ref, o_ref):
  q = q_ref[...]
  t = t_ref[...]
  x32 = _sel4(q & 3, x_ref[...])
  d32 = _sel4(t & 3, d_ref[...])
  hit = d32[:, 5:6] == q.astype(jnp.float32)
  dtile = lax.dot_general(d32[:, 0:4], t1_ref[...], (((1,), (0,)), ((), ())),
                          precision=lax.Precision.HIGHEST,
                          preferred_element_type=jnp.float32)
  xsel = jnp.where(hit, dtile, x32[:, 0:20])
  xb = lax.dot_general(xsel, e_ref[...], (((1,), (0,)), ((), ())),
                       precision=lax.Precision.HIGHEST,
                       preferred_element_type=jnp.float32)
  phase = xb * coef_ref[...] - shift_ref[...]
  o_ref[:, pl.ds(0, NCOLS)] = jnp.cos(phase)
  o_ref[:, pl.ds(NCOLS, FR)] = jnp.where(hit, d32[:, 4:5], x32[:, 20:25])


def _tc_consts():
  dim_t = np.float32(TEMP) ** (
      2.0 * np.floor(np.arange(NPF, dtype=np.float32) / 2.0)
      / np.float32(NPF)).astype(np.float32)
  c = np.arange(NCOLS)
  m32 = c % NPF
  m = np.where(m32 < NPF // 2, m32, m32 - NPF // 2)
  coef = (np.float32(2.0 * np.pi) / dim_t[2 * m]).astype(np.float32)
  shift = np.where(m32 < NPF // 2, np.float32(0.0),
                   np.float32(np.pi / 2)).astype(np.float32)
  e = (c // NPF == np.arange(FR * 4)[:, None]).astype(np.float32)
  t1 = (np.arange(FR * 4)[None, :] % 4 == np.arange(4)[:, None]
        ).astype(np.float32)
  return (e, t1, coef.reshape(1, NCOLS), shift.reshape(1, NCOLS))


def _tc_stage(xout, dout, q, tc):
  e, t1, coef, shift = (jnp.asarray(a) for a in _tc_consts())
  bm = 1024
  return pl.pallas_call(
      _tc_body,
      grid=(BB // bm,),
      in_specs=[
          pl.BlockSpec((bm, 128), lambda i: (i, 0)),
          pl.BlockSpec((bm, 128), lambda i: (i, 0)),
          pl.BlockSpec((bm, 1), lambda i: (i, 0)),
          pl.BlockSpec((bm, 1), lambda i: (i, 0)),
          pl.BlockSpec((FR * 4, NCOLS), lambda i: (0, 0)),
          pl.BlockSpec((4, FR * 4), lambda i: (0, 0)),
          pl.BlockSpec((1, NCOLS), lambda i: (0, 0)),
          pl.BlockSpec((1, NCOLS), lambda i: (0, 0)),
      ],
      out_specs=pl.BlockSpec((bm, NCOLS + FR), lambda i: (i, 0)),
      out_shape=jax.ShapeDtypeStruct((BB, NCOLS + FR), jnp.float32),
  )(xout, dout, q, tc, e, t1, coef, shift)


def kernel(tracklets, tracklet_metadata, detections, write_indices,
           query_indices):
  w = write_indices.astype(jnp.int32)
  q = query_indices.astype(jnp.int32)
  trk_r = tracklets.reshape(MM // 4, 80)
  met_r = tracklet_metadata.reshape(MM // 4, 20)
  z7 = jnp.zeros((MM // 4, 7), jnp.float32)
  comb = jnp.concatenate(
      sum(([trk_r[:, s * 20:(s + 1) * 20], met_r[:, s * 5:(s + 1) * 5], z7]
           for s in range(4)), []), axis=1)
  det2 = jnp.concatenate(
      [detections, w[:, None].astype(jnp.float32),
       jnp.zeros((BB, 26), jnp.float32)], axis=1).reshape(BB // 4, 128)
  xout, dout, tc, _ = _sc_stage()(comb, det2, w, q)
  return _tc_stage(xout, dout, q.reshape(BB, 1), tc.reshape(BB, 1))


# speedup vs baseline: 2.2356x; 2.2356x over previous
import functools

import jax
import jax.numpy as jnp
import numpy as np
from jax import lax
from jax.experimental import pallas as pl
from jax.experimental.pallas import tpu as pltpu
from jax.experimental.pallas import tpu_sc as plsc

FR = 5
NPF = 32
TEMP = 10000.0
MM = 1_000_000
BB = 16384
NC, NS = 2, 16
NW = NC * NS
QW = BB // NW
NCOLS = FR * 4 * NPF
QQ = 128


def _pack_comb_body(t_ref, m_ref, comb_ref):
  t = t_ref[...]
  m = m_ref[...]
  z7 = jnp.zeros((t.shape[0], 7), jnp.float32)
  parts = []
  for s in range(4):
    parts += [t[:, s * 20:(s + 1) * 20], m[:, s * 5:(s + 1) * 5], z7]
  comb_ref[...] = jnp.concatenate(parts, axis=1)


def _pack_det_body(d_ref, det2_ref):
  d = d_ref[...]
  z27 = jnp.zeros((d.shape[0], 27), jnp.float32)
  parts = []
  for s in range(4):
    parts += [d[:, s * 5:(s + 1) * 5], z27]
  det2_ref[...] = jnp.concatenate(parts, axis=1)


def _pack_stage(trk4, met4, det4):
  g = 125
  bm = MM // 4 // g
  comb = pl.pallas_call(
      _pack_comb_body,
      grid=(g,),
      in_specs=[
          pl.BlockSpec((bm, 80), lambda i: (i, 0)),
          pl.BlockSpec((bm, 20), lambda i: (i, 0)),
      ],
      out_specs=pl.BlockSpec((bm, 128), lambda i: (i, 0)),
      out_shape=jax.ShapeDtypeStruct((MM // 4, 128), jnp.float32),
  )(trk4, met4)
  gd = 16
  bd = BB // 4 // gd
  det2 = pl.pallas_call(
      _pack_det_body,
      grid=(gd,),
      in_specs=[pl.BlockSpec((bd, 20), lambda i: (i, 0))],
      out_specs=pl.BlockSpec((bd, 128), lambda i: (i, 0)),
      out_shape=jax.ShapeDtypeStruct((BB // 4, 128), jnp.float32),
  )(det4)
  return comb, det2


_WC = 2048


def _match_body(q_ref, w_ref, t_ref):
  q = q_ref[...]
  t = jnp.full(q.shape, -1, jnp.int32)
  for c in range(BB // _WC):
    wc = w_ref[0, pl.ds(c * _WC, _WC)][None, :]
    jidx = lax.broadcasted_iota(jnp.int32, (q.shape[0], _WC), 1) + c * _WC
    cand = jnp.where(q == wc, jidx, -1)
    t = jnp.maximum(t, jnp.max(cand, axis=1, keepdims=True))
  t_ref[...] = t


def _match_stage(q2, w2):
  bm = 2048
  return pl.pallas_call(
      _match_body,
      grid=(BB // bm,),
      in_specs=[
          pl.BlockSpec((bm, 1), lambda i: (i, 0)),
          pl.BlockSpec((1, BB), lambda i: (0, 0)),
      ],
      out_specs=pl.BlockSpec((bm, 1), lambda i: (i, 0)),
      out_shape=jax.ShapeDtypeStruct((BB, 1), jnp.int32),
  )(q2, w2)


def _sc_body(comb_hbm, det2_hbm, q_hbm, t_hbm,
             xout_hbm, dout_hbm,
             q_v, t_v, qrow_v, trow_v, xq_v, dq_v, sem):
  cid = lax.axis_index("c")
  sid = lax.axis_index("s")
  wid = sid * NC + cid
  qbase = wid * QW
  pltpu.sync_copy(q_hbm.at[pl.ds(qbase, QW)], q_v)
  pltpu.sync_copy(t_hbm.at[pl.ds(qbase, QW)], t_v)
  for j in range(QW // 16):
    s = pl.ds(j * 16, 16)
    tc = jnp.minimum(jnp.maximum(t_v[s], 0), BB - 1)
    qrow_v[s] = lax.shift_right_logical(q_v[s], 2)
    trow_v[s] = lax.shift_right_logical(tc, 2)
  for b in range(QW // QQ):
    sb = pl.ds(b * QQ, QQ)
    cp_x = pltpu.async_copy(comb_hbm.at[qrow_v.at[sb]], xq_v, sem)
    cp_d = pltpu.async_copy(det2_hbm.at[trow_v.at[sb]], dq_v, sem)
    cp_x.wait()
    pltpu.sync_copy(xq_v, xout_hbm.at[pl.ds(qbase + b * QQ, QQ)])
    cp_d.wait()
    pltpu.sync_copy(dq_v, dout_hbm.at[pl.ds(qbase + b * QQ, QQ)])


@functools.lru_cache(maxsize=None)
def _sc_stage():
  mesh = plsc.VectorSubcoreMesh(core_axis_name="c", subcore_axis_name="s",
                                num_cores=NC, num_subcores=NS)
  return pl.kernel(
      _sc_body,
      out_type=(
          jax.ShapeDtypeStruct((BB, 128), jnp.float32),
          jax.ShapeDtypeStruct((BB, 128), jnp.float32),
      ),
      mesh=mesh,
      scratch_types=[
          pltpu.VMEM((QW,), jnp.int32),
          pltpu.VMEM((QW,), jnp.int32),
          pltpu.VMEM((QW,), jnp.int32),
          pltpu.VMEM((QW,), jnp.int32),
          pltpu.VMEM((QQ, 128), jnp.float32),
          pltpu.VMEM((QQ, 128), jnp.float32),
          pltpu.SemaphoreType.DMA,
      ],
  )


def _sel4(m, a):
  return jnp.where(m == 0, a[:, 0:32],
                   jnp.where(m == 1, a[:, 32:64],
                             jnp.where(m == 2, a[:, 64:96], a[:, 96:128])))


def _tc_body(x_ref, d_ref, q_ref, t_ref, e_ref, t1_ref, coef_ref, shift_ref,
             o_ref):
  q = q_ref[...]
  t = t_ref[...]
  tc = jnp.minimum(jnp.maximum(t, 0), BB - 1)
  x32 = _sel4(q & 3, x_ref[...])
  d32 = _sel4(tc & 3, d_ref[...])
  hit = t >= 0
  dtile = lax.dot_general(d32[:, 0:4], t1_ref[...], (((1,), (0,)), ((), ())),
                          precision=lax.Precision.HIGHEST,
                          preferred_element_type=jnp.float32)
  xsel = jnp.where(hit, dtile, x32[:, 0:20])
  xb = lax.dot_general(xsel, e_ref[...], (((1,), (0,)), ((), ())),
                       precision=lax.Precision.HIGHEST,
                       preferred_element_type=jnp.float32)
  phase = xb * coef_ref[...] - shift_ref[...]
  o_ref[:, pl.ds(0, NCOLS)] = jnp.cos(phase)
  o_ref[:, pl.ds(NCOLS, FR)] = jnp.where(hit, d32[:, 4:5], x32[:, 20:25])


def _tc_consts():
  dim_t = np.float32(TEMP) ** (
      2.0 * np.floor(np.arange(NPF, dtype=np.float32) / 2.0)
      / np.float32(NPF)).astype(np.float32)
  c = np.arange(NCOLS)
  m32 = c % NPF
  m = np.where(m32 < NPF // 2, m32, m32 - NPF // 2)
  coef = (np.float32(2.0 * np.pi) / dim_t[2 * m]).astype(np.float32)
  shift = np.where(m32 < NPF // 2, np.float32(0.0),
                   np.float32(np.pi / 2)).astype(np.float32)
  e = (c // NPF == np.arange(FR * 4)[:, None]).astype(np.float32)
  t1 = (np.arange(FR * 4)[None, :] % 4 == np.arange(4)[:, None]
        ).astype(np.float32)
  return (e, t1, coef.reshape(1, NCOLS), shift.reshape(1, NCOLS))


def _tc_stage(xout, dout, q2, t2):
  e, t1, coef, shift = (jnp.asarray(a) for a in _tc_consts())
  bm = 1024
  return pl.pallas_call(
      _tc_body,
      grid=(BB // bm,),
      in_specs=[
          pl.BlockSpec((bm, 128), lambda i: (i, 0)),
          pl.BlockSpec((bm, 128), lambda i: (i, 0)),
          pl.BlockSpec((bm, 1), lambda i: (i, 0)),
          pl.BlockSpec((bm, 1), lambda i: (i, 0)),
          pl.BlockSpec((FR * 4, NCOLS), lambda i: (0, 0)),
          pl.BlockSpec((4, FR * 4), lambda i: (0, 0)),
          pl.BlockSpec((1, NCOLS), lambda i: (0, 0)),
          pl.BlockSpec((1, NCOLS), lambda i: (0, 0)),
      ],
      out_specs=pl.BlockSpec((bm, NCOLS + FR), lambda i: (i, 0)),
      out_shape=jax.ShapeDtypeStruct((BB, NCOLS + FR), jnp.float32),
  )(xout, dout, q2, t2, e, t1, coef, shift)


def kernel(tracklets, tracklet_metadata, detections, write_indices,
           query_indices):
  w = write_indices.astype(jnp.int32)
  q = query_indices.astype(jnp.int32)
  comb, det2 = _pack_stage(tracklets.reshape(MM // 4, 80),
                           tracklet_metadata.reshape(MM // 4, 20),
                           detections.reshape(BB // 4, 20))
  t2 = _match_stage(q.reshape(BB, 1), w.reshape(1, BB))
  xout, dout = _sc_stage()(comb, det2, q, t2.reshape(BB))
  return _tc_stage(xout, dout, q.reshape(BB, 1), t2)


# speedup vs baseline: 4.3875x; 1.9626x over previous
import functools

import jax
import jax.numpy as jnp
import numpy as np
from jax import lax
from jax.experimental import pallas as pl
from jax.experimental.pallas import tpu as pltpu
from jax.experimental.pallas import tpu_sc as plsc

FR = 5
NPF = 32
TEMP = 10000.0
MM = 1_000_000
BB = 16384
NC, NS = 2, 16
NW = NC * NS
QW = BB // NW
NCOLS = FR * 4 * NPF
QQ = 128


def _pack_perm():
  ms = []
  for c in range(4):
    m = np.zeros((20, 128), np.float32)
    for s in range(4):
      for f in range(FR):
        m[s * 5 + f, s * 32 + f * 4 + c] = 1.0
    ms.append(m)
  mm = np.zeros((20, 128), np.float32)
  for s in range(4):
    for f in range(FR):
      mm[s * 5 + f, s * 32 + 20 + f] = 1.0
  ms.append(mm)
  return np.stack(ms)


def _pack_comb_body(p0, p1, p2, p3, m_ref, perm_ref, comb_ref):
  acc = lax.dot_general(p0[...], perm_ref[0], (((1,), (0,)), ((), ())),
                        precision=lax.Precision.HIGHEST,
                        preferred_element_type=jnp.float32)
  for i, p in enumerate((p1, p2, p3, m_ref)):
    acc = acc + lax.dot_general(p[...], perm_ref[i + 1],
                                (((1,), (0,)), ((), ())),
                                precision=lax.Precision.HIGHEST,
                                preferred_element_type=jnp.float32)
  comb_ref[...] = acc


def _pack_det_body(d_ref, det2_ref):
  d = d_ref[...]
  z27 = jnp.zeros((d.shape[0], 27), jnp.float32)
  parts = []
  for s in range(4):
    parts += [d[:, s * 5:(s + 1) * 5], z27]
  det2_ref[...] = jnp.concatenate(parts, axis=1)


def _pack_stage(planes, met4, det4):
  perm = jnp.asarray(_pack_perm())
  g = 125
  bm = MM // 4 // g
  comb = pl.pallas_call(
      _pack_comb_body,
      grid=(g,),
      in_specs=[pl.BlockSpec((bm, 20), lambda i: (i, 0)) for _ in range(4)]
      + [
          pl.BlockSpec((bm, 20), lambda i: (i, 0)),
          pl.BlockSpec((5, 20, 128), lambda i: (0, 0, 0)),
      ],
      out_specs=pl.BlockSpec((bm, 128), lambda i: (i, 0)),
      out_shape=jax.ShapeDtypeStruct((MM // 4, 128), jnp.float32),
  )(*planes, met4, perm)
  gd = 16
  bd = BB // 4 // gd
  det2 = pl.pallas_call(
      _pack_det_body,
      grid=(gd,),
      in_specs=[pl.BlockSpec((bd, 20), lambda i: (i, 0))],
      out_specs=pl.BlockSpec((bd, 128), lambda i: (i, 0)),
      out_shape=jax.ShapeDtypeStruct((BB // 4, 128), jnp.float32),
  )(det4)
  return comb, det2


_WC = 2048


def _match_body(q_ref, w_ref, t_ref):
  q = q_ref[...]
  t = jnp.full(q.shape, -1, jnp.int32)
  for c in range(BB // _WC):
    wc = w_ref[0, pl.ds(c * _WC, _WC)][None, :]
    jidx = lax.broadcasted_iota(jnp.int32, (q.shape[0], _WC), 1) + c * _WC
    cand = jnp.where(q == wc, jidx, -1)
    t = jnp.maximum(t, jnp.max(cand, axis=1, keepdims=True))
  t_ref[...] = t


def _match_stage(q2, w2):
  bm = 2048
  return pl.pallas_call(
      _match_body,
      grid=(BB // bm,),
      in_specs=[
          pl.BlockSpec((bm, 1), lambda i: (i, 0)),
          pl.BlockSpec((1, BB), lambda i: (0, 0)),
      ],
      out_specs=pl.BlockSpec((bm, 1), lambda i: (i, 0)),
      out_shape=jax.ShapeDtypeStruct((BB, 1), jnp.int32),
  )(q2, w2)


def _sc_body(comb_hbm, det2_hbm, q_hbm, t_hbm,
             xout_hbm, dout_hbm,
             q_v, t_v, qrow_v, trow_v, xq_v, dq_v, sem):
  cid = lax.axis_index("c")
  sid = lax.axis_index("s")
  wid = sid * NC + cid
  qbase = wid * QW
  pltpu.sync_copy(q_hbm.at[pl.ds(qbase, QW)], q_v)
  pltpu.sync_copy(t_hbm.at[pl.ds(qbase, QW)], t_v)
  for j in range(QW // 16):
    s = pl.ds(j * 16, 16)
    tc = jnp.minimum(jnp.maximum(t_v[s], 0), BB - 1)
    qrow_v[s] = lax.shift_right_logical(q_v[s], 2)
    trow_v[s] = lax.shift_right_logical(tc, 2)
  for b in range(QW // QQ):
    sb = pl.ds(b * QQ, QQ)
    cp_x = pltpu.async_copy(comb_hbm.at[qrow_v.at[sb]], xq_v, sem)
    cp_d = pltpu.async_copy(det2_hbm.at[trow_v.at[sb]], dq_v, sem)
    cp_x.wait()
    pltpu.sync_copy(xq_v, xout_hbm.at[pl.ds(qbase + b * QQ, QQ)])
    cp_d.wait()
    pltpu.sync_copy(dq_v, dout_hbm.at[pl.ds(qbase + b * QQ, QQ)])


@functools.lru_cache(maxsize=None)
def _sc_stage():
  mesh = plsc.VectorSubcoreMesh(core_axis_name="c", subcore_axis_name="s",
                                num_cores=NC, num_subcores=NS)
  return pl.kernel(
      _sc_body,
      out_type=(
          jax.ShapeDtypeStruct((BB, 128), jnp.float32),
          jax.ShapeDtypeStruct((BB, 128), jnp.float32),
      ),
      mesh=mesh,
      scratch_types=[
          pltpu.VMEM((QW,), jnp.int32),
          pltpu.VMEM((QW,), jnp.int32),
          pltpu.VMEM((QW,), jnp.int32),
          pltpu.VMEM((QW,), jnp.int32),
          pltpu.VMEM((QQ, 128), jnp.float32),
          pltpu.VMEM((QQ, 128), jnp.float32),
          pltpu.SemaphoreType.DMA,
      ],
  )


def _sel4(m, a):
  return jnp.where(m == 0, a[:, 0:32],
                   jnp.where(m == 1, a[:, 32:64],
                             jnp.where(m == 2, a[:, 64:96], a[:, 96:128])))


def _tc_body(x_ref, d_ref, q_ref, t_ref, e_ref, t1_ref, coef_ref, shift_ref,
             o_ref):
  q = q_ref[...]
  t = t_ref[...]
  tc = jnp.minimum(jnp.maximum(t, 0), BB - 1)
  x32 = _sel4(q & 3, x_ref[...])
  d32 = _sel4(tc & 3, d_ref[...])
  hit = t >= 0
  dtile = lax.dot_general(d32[:, 0:4], t1_ref[...], (((1,), (0,)), ((), ())),
                          precision=lax.Precision.HIGHEST,
                          preferred_element_type=jnp.float32)
  xsel = jnp.where(hit, dtile, x32[:, 0:20])
  xb = lax.dot_general(xsel, e_ref[...], (((1,), (0,)), ((), ())),
                       precision=lax.Precision.HIGHEST,
                       preferred_element_type=jnp.float32)
  phase = xb * coef_ref[...] - shift_ref[...]
  o_ref[:, pl.ds(0, NCOLS)] = jnp.cos(phase)
  o_ref[:, pl.ds(NCOLS, FR)] = jnp.where(hit, d32[:, 4:5], x32[:, 20:25])


def _tc_consts():
  dim_t = np.float32(TEMP) ** (
      2.0 * np.floor(np.arange(NPF, dtype=np.float32) / 2.0)
      / np.float32(NPF)).astype(np.float32)
  c = np.arange(NCOLS)
  m32 = c % NPF
  m = np.where(m32 < NPF // 2, m32, m32 - NPF // 2)
  coef = (np.float32(2.0 * np.pi) / dim_t[2 * m]).astype(np.float32)
  shift = np.where(m32 < NPF // 2, np.float32(0.0),
                   np.float32(np.pi / 2)).astype(np.float32)
  e = (c // NPF == np.arange(FR * 4)[:, None]).astype(np.float32)
  t1 = (np.arange(FR * 4)[None, :] % 4 == np.arange(4)[:, None]
        ).astype(np.float32)
  return (e, t1, coef.reshape(1, NCOLS), shift.reshape(1, NCOLS))


def _tc_stage(xout, dout, q2, t2):
  e, t1, coef, shift = (jnp.asarray(a) for a in _tc_consts())
  bm = 1024
  return pl.pallas_call(
      _tc_body,
      grid=(BB // bm,),
      in_specs=[
          pl.BlockSpec((bm, 128), lambda i: (i, 0)),
          pl.BlockSpec((bm, 128), lambda i: (i, 0)),
          pl.BlockSpec((bm, 1), lambda i: (i, 0)),
          pl.BlockSpec((bm, 1), lambda i: (i, 0)),
          pl.BlockSpec((FR * 4, NCOLS), lambda i: (0, 0)),
          pl.BlockSpec((4, FR * 4), lambda i: (0, 0)),
          pl.BlockSpec((1, NCOLS), lambda i: (0, 0)),
          pl.BlockSpec((1, NCOLS), lambda i: (0, 0)),
      ],
      out_specs=pl.BlockSpec((bm, NCOLS + FR), lambda i: (i, 0)),
      out_shape=jax.ShapeDtypeStruct((BB, NCOLS + FR), jnp.float32),
  )(xout, dout, q2, t2, e, t1, coef, shift)


def kernel(tracklets, tracklet_metadata, detections, write_indices,
           query_indices):
  w = write_indices.astype(jnp.int32)
  q = query_indices.astype(jnp.int32)
  planes = [tracklets[:, :, c].reshape(MM // 4, 20) for c in range(4)]
  met4 = tracklet_metadata.reshape(MM // 4, 20)
  comb, det2 = _pack_stage(planes, met4, detections.reshape(BB // 4, 20))
  t2 = _match_stage(q.reshape(BB, 1), w.reshape(1, BB))
  xout, dout = _sc_stage()(comb, det2, q, t2.reshape(BB))
  return _tc_stage(xout, dout, q.reshape(BB, 1), t2)


# speedup vs baseline: 4.6202x; 1.0530x over previous
import functools

import jax
import jax.numpy as jnp
import numpy as np
from jax import lax
from jax.experimental import pallas as pl
from jax.experimental.pallas import tpu as pltpu
from jax.experimental.pallas import tpu_sc as plsc

FR = 5
NPF = 32
TEMP = 10000.0
MM = 1_000_000
BB = 16384
NC, NS = 2, 16
NW = NC * NS
QW = BB // NW
NCOLS = FR * 4 * NPF
QQ = 128


def _pack_perm():
  m = np.zeros((100, 128), np.float32)
  for c in range(4):
    for s in range(4):
      for f in range(FR):
        m[c * 20 + s * 5 + f, s * 32 + f * 4 + c] = 1.0
  for s in range(4):
    for f in range(FR):
      m[80 + s * 5 + f, s * 32 + 20 + f] = 1.0
  return m


def _pack_comb_body(p0, p1, p2, p3, m_ref, perm_ref, comb_ref):
  x = jnp.concatenate([p0[...], p1[...], p2[...], p3[...], m_ref[...]],
                      axis=1)
  comb_ref[...] = lax.dot_general(x, perm_ref[...], (((1,), (0,)), ((), ())),
                                  precision=lax.Precision.HIGHEST,
                                  preferred_element_type=jnp.float32)


def _pack_det_body(d_ref, det2_ref):
  d = d_ref[...]
  z27 = jnp.zeros((d.shape[0], 27), jnp.float32)
  parts = []
  for s in range(4):
    parts += [d[:, s * 5:(s + 1) * 5], z27]
  det2_ref[...] = jnp.concatenate(parts, axis=1)


def _pack_stage(planes, met4, det4):
  perm = jnp.asarray(_pack_perm())
  g = 125
  bm = MM // 4 // g
  comb = pl.pallas_call(
      _pack_comb_body,
      grid=(g,),
      in_specs=[pl.BlockSpec((bm, 20), lambda i: (i, 0)) for _ in range(4)]
      + [
          pl.BlockSpec((bm, 20), lambda i: (i, 0)),
          pl.BlockSpec((100, 128), lambda i: (0, 0)),
      ],
      out_specs=pl.BlockSpec((bm, 128), lambda i: (i, 0)),
      out_shape=jax.ShapeDtypeStruct((MM // 4, 128), jnp.float32),
  )(*planes, met4, perm)
  gd = 16
  bd = BB // 4 // gd
  det2 = pl.pallas_call(
      _pack_det_body,
      grid=(gd,),
      in_specs=[pl.BlockSpec((bd, 20), lambda i: (i, 0))],
      out_specs=pl.BlockSpec((bd, 128), lambda i: (i, 0)),
      out_shape=jax.ShapeDtypeStruct((BB // 4, 128), jnp.float32),
  )(det4)
  return comb, det2


_WC = 2048


def _match_body(q_ref, w_ref, t_ref):
  q = q_ref[...]
  t = jnp.full(q.shape, -1, jnp.int32)
  for c in range(BB // _WC):
    wc = w_ref[0, pl.ds(c * _WC, _WC)][None, :]
    jidx = lax.broadcasted_iota(jnp.int32, (q.shape[0], _WC), 1) + c * _WC
    cand = jnp.where(q == wc, jidx, -1)
    t = jnp.maximum(t, jnp.max(cand, axis=1, keepdims=True))
  t_ref[...] = t


def _match_stage(q2, w2):
  bm = 2048
  return pl.pallas_call(
      _match_body,
      grid=(BB // bm,),
      in_specs=[
          pl.BlockSpec((bm, 1), lambda i: (i, 0)),
          pl.BlockSpec((1, BB), lambda i: (0, 0)),
      ],
      out_specs=pl.BlockSpec((bm, 1), lambda i: (i, 0)),
      out_shape=jax.ShapeDtypeStruct((BB, 1), jnp.int32),
  )(q2, w2)


def _sc_body(comb_hbm, det2_hbm, q_hbm, t_hbm,
             xout_hbm, dout_hbm,
             q_v, t_v, qrow_v, trow_v, xq_v, dq_v, sem):
  cid = lax.axis_index("c")
  sid = lax.axis_index("s")
  wid = sid * NC + cid
  qbase = wid * QW
  pltpu.sync_copy(q_hbm.at[pl.ds(qbase, QW)], q_v)
  pltpu.sync_copy(t_hbm.at[pl.ds(qbase, QW)], t_v)
  for j in range(QW // 16):
    s = pl.ds(j * 16, 16)
    tc = jnp.minimum(jnp.maximum(t_v[s], 0), BB - 1)
    qrow_v[s] = lax.shift_right_logical(q_v[s], 2)
    trow_v[s] = lax.shift_right_logical(tc, 2)
  for b in range(QW // QQ):
    sb = pl.ds(b * QQ, QQ)
    cp_x = pltpu.async_copy(comb_hbm.at[qrow_v.at[sb]], xq_v, sem)
    cp_d = pltpu.async_copy(det2_hbm.at[trow_v.at[sb]], dq_v, sem)
    cp_x.wait()
    pltpu.sync_copy(xq_v, xout_hbm.at[pl.ds(qbase + b * QQ, QQ)])
    cp_d.wait()
    pltpu.sync_copy(dq_v, dout_hbm.at[pl.ds(qbase + b * QQ, QQ)])


@functools.lru_cache(maxsize=None)
def _sc_stage():
  mesh = plsc.VectorSubcoreMesh(core_axis_name="c", subcore_axis_name="s",
                                num_cores=NC, num_subcores=NS)
  return pl.kernel(
      _sc_body,
      out_type=(
          jax.ShapeDtypeStruct((BB, 128), jnp.float32),
          jax.ShapeDtypeStruct((BB, 128), jnp.float32),
      ),
      mesh=mesh,
      scratch_types=[
          pltpu.VMEM((QW,), jnp.int32),
          pltpu.VMEM((QW,), jnp.int32),
          pltpu.VMEM((QW,), jnp.int32),
          pltpu.VMEM((QW,), jnp.int32),
          pltpu.VMEM((QQ, 128), jnp.float32),
          pltpu.VMEM((QQ, 128), jnp.float32),
          pltpu.SemaphoreType.DMA,
      ],
  )


def _sel4(m, a):
  return jnp.where(m == 0, a[:, 0:32],
                   jnp.where(m == 1, a[:, 32:64],
                             jnp.where(m == 2, a[:, 64:96], a[:, 96:128])))


def _tc_body(x_ref, d_ref, q_ref, t_ref, e_ref, t1_ref, coef_ref, shift_ref,
             o_ref):
  q = q_ref[...]
  t = t_ref[...]
  tc = jnp.minimum(jnp.maximum(t, 0), BB - 1)
  x32 = _sel4(q & 3, x_ref[...])
  d32 = _sel4(tc & 3, d_ref[...])
  hit = t >= 0
  dtile = lax.dot_general(d32[:, 0:4], t1_ref[...], (((1,), (0,)), ((), ())),
                          precision=lax.Precision.HIGHEST,
                          preferred_element_type=jnp.float32)
  xsel = jnp.where(hit, dtile, x32[:, 0:20])
  xb = lax.dot_general(xsel, e_ref[...], (((1,), (0,)), ((), ())),
                       precision=lax.Precision.HIGHEST,
                       preferred_element_type=jnp.float32)
  phase = xb * coef_ref[...] - shift_ref[...]
  o_ref[:, pl.ds(0, NCOLS)] = jnp.cos(phase)
  o_ref[:, pl.ds(NCOLS, FR)] = jnp.where(hit, d32[:, 4:5], x32[:, 20:25])


def _tc_consts():
  dim_t = np.float32(TEMP) ** (
      2.0 * np.floor(np.arange(NPF, dtype=np.float32) / 2.0)
      / np.float32(NPF)).astype(np.float32)
  c = np.arange(NCOLS)
  m32 = c % NPF
  m = np.where(m32 < NPF // 2, m32, m32 - NPF // 2)
  coef = (np.float32(2.0 * np.pi) / dim_t[2 * m]).astype(np.float32)
  shift = np.where(m32 < NPF // 2, np.float32(0.0),
                   np.float32(np.pi / 2)).astype(np.float32)
  e = (c // NPF == np.arange(FR * 4)[:, None]).astype(np.float32)
  t1 = (np.arange(FR * 4)[None, :] % 4 == np.arange(4)[:, None]
        ).astype(np.float32)
  return (e, t1, coef.reshape(1, NCOLS), shift.reshape(1, NCOLS))


def _tc_stage(xout, dout, q2, t2):
  e, t1, coef, shift = (jnp.asarray(a) for a in _tc_consts())
  bm = 1024
  return pl.pallas_call(
      _tc_body,
      grid=(BB // bm,),
      in_specs=[
          pl.BlockSpec((bm, 128), lambda i: (i, 0)),
          pl.BlockSpec((bm, 128), lambda i: (i, 0)),
          pl.BlockSpec((bm, 1), lambda i: (i, 0)),
          pl.BlockSpec((bm, 1), lambda i: (i, 0)),
          pl.BlockSpec((FR * 4, NCOLS), lambda i: (0, 0)),
          pl.BlockSpec((4, FR * 4), lambda i: (0, 0)),
          pl.BlockSpec((1, NCOLS), lambda i: (0, 0)),
          pl.BlockSpec((1, NCOLS), lambda i: (0, 0)),
      ],
      out_specs=pl.BlockSpec((bm, NCOLS + FR), lambda i: (i, 0)),
      out_shape=jax.ShapeDtypeStruct((BB, NCOLS + FR), jnp.float32),
  )(xout, dout, q2, t2, e, t1, coef, shift)


def kernel(tracklets, tracklet_metadata, detections, write_indices,
           query_indices):
  w = write_indices.astype(jnp.int32)
  q = query_indices.astype(jnp.int32)
  planes = [tracklets[:, :, c].reshape(MM // 4, 20) for c in range(4)]
  met4 = tracklet_metadata.reshape(MM // 4, 20)
  comb, det2 = _pack_stage(planes, met4, detections.reshape(BB // 4, 20))
  t2 = _match_stage(q.reshape(BB, 1), w.reshape(1, BB))
  xout, dout = _sc_stage()(comb, det2, q, t2.reshape(BB))
  return _tc_stage(xout, dout, q.reshape(BB, 1), t2)
